# baseline (speedup 1.0000x reference)
import functools

import jax
import jax.numpy as jnp
from jax.experimental import pallas as pl
from jax.experimental.pallas import tpu as pltpu


def _se_fused_kernel(x_ref, z_ref, w1t_ref, b1_ref, w2t_ref, b2_ref, o_ref,
                     *, inv_hw):
    pooled = jnp.sum(x_ref[...].astype(jnp.float32), axis=(1, 2)) * inv_hw
    pooled = pooled.reshape(1, -1)
    h = jnp.dot(pooled, w1t_ref[...],
                preferred_element_type=jnp.float32) + b1_ref[...]
    h = h * jax.nn.sigmoid(h)
    y = jnp.dot(h, w2t_ref[...],
                preferred_element_type=jnp.float32) + b2_ref[...]
    s = jax.nn.sigmoid(y)
    s3 = s.reshape(-1, 1, 1)
    o_ref[...] = (s3 * z_ref[...].astype(jnp.float32)).astype(o_ref.dtype)


def _se_call(x3, z3, w1t, b1r, w2t, b2r, *, inp, oup, inv_hw):
    rows, H, W = x3.shape
    nb = rows // inp
    _, Hz, Wz = z3.shape
    sq = w1t.shape[1]
    return pl.pallas_call(
        functools.partial(_se_fused_kernel, inv_hw=inv_hw),
        out_shape=jax.ShapeDtypeStruct(z3.shape, z3.dtype),
        grid=(nb,),
        in_specs=[
            pl.BlockSpec((inp, H, W), lambda b: (b, 0, 0)),
            pl.BlockSpec((oup, Hz, Wz), lambda b: (b, 0, 0)),
            pl.BlockSpec((inp, sq), lambda b: (0, 0)),
            pl.BlockSpec((1, sq), lambda b: (0, 0)),
            pl.BlockSpec((sq, oup), lambda b: (0, 0)),
            pl.BlockSpec((1, oup), lambda b: (0, 0)),
        ],
        out_specs=pl.BlockSpec((oup, Hz, Wz), lambda b: (b, 0, 0)),
        compiler_params=pltpu.CompilerParams(
            dimension_semantics=("parallel",),
            vmem_limit_bytes=56 * 1024 * 1024),
    )(x3, z3, w1t, b1r, w2t, b2r)


def kernel(x, z, w1, b1, w2, b2):
    B, inp, H, W = x.shape
    Bz, oup, Hz, Wz = z.shape
    assert B == Bz
    sq = w1.shape[0]

    x3 = x.reshape(B * inp, H, W)
    z3 = z.reshape(B * oup, Hz, Wz)
    w1t = w1.astype(jnp.float32).T
    w2t = w2.astype(jnp.float32).T
    b1r = b1.astype(jnp.float32).reshape(1, sq)
    b2r = b2.astype(jnp.float32).reshape(1, oup)

    call = functools.partial(_se_call, inp=inp, oup=oup,
                             inv_hw=float(1.0 / (H * W)))

    devs = jax.devices()
    ndev = len(devs) if devs and devs[0].platform == "tpu" else 1
    if ndev > 1 and B % ndev == 0:
        mesh = jax.sharding.Mesh(devs[:ndev], ("d",))
        P = jax.sharding.PartitionSpec
        out3 = jax.shard_map(
            call, mesh=mesh,
            in_specs=(P("d"), P("d"), P(), P(), P(), P()),
            out_specs=P("d"), check_vma=False,
        )(x3, z3, w1t, b1r, w2t, b2r)
    else:
        out3 = call(x3, z3, w1t, b1r, w2t, b2r)

    return out3.reshape(B, oup, Hz, Wz)


# speedup vs baseline: 1.7171x; 1.7171x over previous
import functools

import jax
import jax.numpy as jnp
from jax.experimental import pallas as pl
from jax.experimental.pallas import tpu as pltpu


def _se_kernel(x_lo_ref, x_hi_ref, z_lo_ref, z_hi_ref,
               w1t_ref, b1r_ref, w2t_ref, b2r_ref,
               o_hbm, obuf, sems, *, inv_hw, nb, oup):
    b = pl.program_id(0)
    buf = jax.lax.rem(b, 2)
    half = oup // 2

    @pl.when(b >= 2)
    def _():
        for h in range(2):
            pltpu.make_async_copy(
                obuf.at[buf, pl.ds(h * half, half)],
                obuf.at[buf, pl.ds(h * half, half)],
                sems.at[buf, h]).wait()

    pooled_lo = jnp.sum(x_lo_ref[...].astype(jnp.float32), axis=(1, 2))
    pooled_hi = jnp.sum(x_hi_ref[...].astype(jnp.float32), axis=(1, 2))
    pooled = jnp.concatenate([pooled_lo, pooled_hi]).reshape(1, -1) * inv_hw
    h1 = jnp.dot(pooled, w1t_ref[...],
                 preferred_element_type=jnp.float32) + b1r_ref[...]
    h1 = h1 * jax.nn.sigmoid(h1)
    y = jnp.dot(h1, w2t_ref[...],
                preferred_element_type=jnp.float32) + b2r_ref[...]
    s = jax.nn.sigmoid(y)

    s_lo = s[:, :half].reshape(half, 1, 1)
    s_hi = s[:, half:].reshape(half, 1, 1)
    obuf[buf, pl.ds(0, half)] = (s_lo * z_lo_ref[...].astype(jnp.float32)
                                 ).astype(obuf.dtype)
    obuf[buf, pl.ds(half, half)] = (s_hi * z_hi_ref[...].astype(jnp.float32)
                                    ).astype(obuf.dtype)

    for h in range(2):
        pltpu.make_async_copy(
            obuf.at[buf, pl.ds(h * half, half)],
            o_hbm.at[pl.ds(b * oup + h * half, half)],
            sems.at[buf, h]).start()

    @pl.when(b == nb - 1)
    def _():
        for bb in range(2 if nb > 1 else 1):
            for h in range(2):
                pltpu.make_async_copy(
                    obuf.at[bb, pl.ds(h * half, half)],
                    obuf.at[bb, pl.ds(h * half, half)],
                    sems.at[bb, h]).wait()


def kernel(x, z, w1, b1, w2, b2):
    B, inp, H, W = x.shape
    Bz, oup, Hz, Wz = z.shape
    assert B == Bz
    sq = w1.shape[0]
    hin = inp // 2

    x3 = x.reshape(B * inp, H, W)
    z3 = z.reshape(B * oup, Hz, Wz)
    w1t = w1.astype(jnp.float32).T
    w2t = w2.astype(jnp.float32).T
    b1r = b1.astype(jnp.float32).reshape(1, sq)
    b2r = b2.astype(jnp.float32).reshape(1, oup)

    out3 = pl.pallas_call(
        functools.partial(_se_kernel, inv_hw=float(1.0 / (H * W)),
                          nb=B, oup=oup),
        out_shape=jax.ShapeDtypeStruct((B * oup, Hz, Wz), z.dtype),
        grid=(B,),
        in_specs=[
            pl.BlockSpec((hin, H, W), lambda b: (2 * b, 0, 0)),
            pl.BlockSpec((hin, H, W), lambda b: (2 * b + 1, 0, 0)),
            pl.BlockSpec((oup // 2, Hz, Wz), lambda b: (2 * b, 0, 0)),
            pl.BlockSpec((oup // 2, Hz, Wz), lambda b: (2 * b + 1, 0, 0)),
            pl.BlockSpec((inp, sq), lambda b: (0, 0)),
            pl.BlockSpec((1, sq), lambda b: (0, 0)),
            pl.BlockSpec((sq, oup), lambda b: (0, 0)),
            pl.BlockSpec((1, oup), lambda b: (0, 0)),
        ],
        out_specs=pl.BlockSpec(memory_space=pl.ANY),
        scratch_shapes=[
            pltpu.VMEM((2, oup, Hz, Wz), z.dtype),
            pltpu.SemaphoreType.DMA((2, 2)),
        ],
        compiler_params=pltpu.CompilerParams(
            dimension_semantics=("arbitrary",),
            vmem_limit_bytes=58 * 1024 * 1024),
    )(x3, x3, z3, z3, w1t, b1r, w2t, b2r)

    return out3.reshape(B, oup, Hz, Wz)
